# baseline (speedup 1.0000x reference)
import jax
import jax.numpy as jnp
from jax import lax
from jax.experimental import pallas as pl
from jax.experimental.pallas import tpu as pltpu
from jax.experimental.pallas import tpu_sc as plsc

NUM_ROWS = 1000001
DIM = 32
B_TOTAL = 4096 * 200
NC, NS = 2, 16
NW = NC * NS
BLOCK = 1280
N_BLOCKS = B_TOTAL // (NW * BLOCK)
B_PER_W = N_BLOCKS * BLOCK
NBUF = 2
N_GROUPS = N_BLOCKS // NBUF


def _body(idx_hbm, table_hbm, out_hbm, idx_v, *scratch):
    bufs = scratch[:NBUF]
    sems = scratch[NBUF:]
    wid = lax.axis_index("s") * NC + lax.axis_index("c")
    base = wid * B_PER_W
    pltpu.sync_copy(idx_hbm.at[wid], idx_v)

    for b in range(NBUF):
        pltpu.make_async_copy(table_hbm.at[idx_v.at[b]], bufs[b], sems[b]).start()

    def group(g):
        k0 = g * NBUF
        for b in range(NBUF):
            k = k0 + b
            pltpu.make_async_copy(
                table_hbm.at[pl.ds(0, BLOCK)], bufs[b], sems[b]
            ).wait()
            pltpu.sync_copy(bufs[b], out_hbm.at[pl.ds(base + k * BLOCK, BLOCK)])
            nxt = k + NBUF

            @pl.when(nxt < N_BLOCKS)
            def _():
                pltpu.make_async_copy(
                    table_hbm.at[idx_v.at[nxt]], bufs[b], sems[b]
                ).start()

    pl.loop(0, N_GROUPS)(group)


@jax.jit
def _gather(indices_blocked, weight):
    mesh = plsc.VectorSubcoreMesh(core_axis_name="c", subcore_axis_name="s")
    flat = pl.kernel(
        _body,
        out_type=jax.ShapeDtypeStruct((B_TOTAL, DIM), jnp.float32),
        mesh=mesh,
        scratch_types=[pltpu.VMEM((N_BLOCKS, BLOCK), jnp.int32)]
        + [pltpu.VMEM((BLOCK, DIM), jnp.float32) for _ in range(NBUF)]
        + [pltpu.SemaphoreType.DMA for _ in range(NBUF)],
        compiler_params=pltpu.CompilerParams(use_tc_tiling_on_sc=False),
    )(indices_blocked, weight)
    return flat


def kernel(indices, weight):
    idx_blocked = indices.reshape(NW, N_BLOCKS, BLOCK)
    flat = _gather(idx_blocked, weight)
    return flat.reshape(indices.shape + (DIM,))
